# Initial kernel scaffold; baseline (speedup 1.0000x reference)
import jax
import jax.numpy as jnp
from jax.experimental import pallas as pl


def kernel(x, synset_indices, synset_values):
    raise NotImplementedError("write your pallas kernel here")


# speedup vs baseline: 5.0413x; 5.0413x over previous
import functools

import jax
import jax.numpy as jnp
from jax import lax
from jax.experimental import pallas as pl
from jax.experimental.pallas import tpu as pltpu
from jax.experimental.pallas import tpu_sc as plsc

N_NODES = 10000
N_EDGES = 320000
D_FEAT = 128

NC = 2
NS = 16
NW = NC * NS
CHUNK = 128
EDGES_PER_TILE = N_EDGES // NW
NCHUNK = -(-EDGES_PER_TILE // CHUNK)
EPT_PAD = NCHUNK * CHUNK
N_PAD = 10240
ROWS_PER_SUB = N_PAD // NS


@functools.partial(
    pl.kernel,
    out_type=jax.ShapeDtypeStruct((NC, N_PAD, D_FEAT), jnp.float32),
    mesh=plsc.VectorSubcoreMesh(core_axis_name="c", subcore_axis_name="s"),
    scratch_types=[
        pltpu.VMEM((NCHUNK, CHUNK), jnp.int32),
        pltpu.VMEM((NCHUNK, CHUNK), jnp.int32),
        pltpu.VMEM((NCHUNK, CHUNK), jnp.float32),
        pltpu.VMEM((CHUNK, D_FEAT), jnp.float32),
        pltpu.VMEM_SHARED((N_PAD, D_FEAT), jnp.float32),
        pltpu.SemaphoreType.DMA,
    ],
)
def _sc_spmm(xt_hbm, col_hbm, row_hbm, val_hbm, part_hbm,
             col_v, row_v, val_v, rows_v, acc, sem):
    cid = lax.axis_index("c")
    sid = lax.axis_index("s")
    tid = cid * NS + sid

    def zbody(r, carry):
        for j in range(D_FEAT // 16):
            rows_v[r, pl.ds(j * 16, 16)] = jnp.zeros((16,), jnp.float32)
        return carry

    lax.fori_loop(0, CHUNK, zbody, 0)
    for t in range(ROWS_PER_SUB // CHUNK):
        pltpu.sync_copy(rows_v, acc.at[pl.ds(sid * ROWS_PER_SUB + t * CHUNK, CHUNK)])
    plsc.subcore_barrier()

    pltpu.sync_copy(col_hbm.at[tid], col_v)
    pltpu.sync_copy(row_hbm.at[tid], row_v)
    pltpu.sync_copy(val_hbm.at[tid], val_v)

    def chunk_body(k, carry):
        pltpu.async_copy(xt_hbm.at[col_v.at[k]], rows_v, sem).wait()

        def scale(g, c2):
            vv = val_v[k, pl.ds(g * 16, 16)]
            base = g * 16
            for i in range(16):
                v = vv[i]
                for j in range(D_FEAT // 16):
                    sl = pl.ds(j * 16, 16)
                    rows_v[base + i, sl] = rows_v[base + i, sl] * v
            return c2

        lax.fori_loop(0, CHUNK // 16, scale, 0)

        pltpu.sync_copy(rows_v, acc.at[row_v.at[k]], add=True)
        return carry

    lax.fori_loop(0, NCHUNK, chunk_body, 0)
    plsc.subcore_barrier()

    base = sid * ROWS_PER_SUB
    pltpu.sync_copy(acc.at[pl.ds(base, ROWS_PER_SUB)],
                    part_hbm.at[cid].at[pl.ds(base, ROWS_PER_SUB)])


_BN = 1024


def _merge_body(p_ref, o_ref):
    s = p_ref[0] + p_ref[1]
    o_ref[...] = s.T


_merge = pl.pallas_call(
    _merge_body,
    grid=(N_PAD // _BN,),
    in_specs=[pl.BlockSpec((NC, _BN, D_FEAT), lambda i: (0, i, 0))],
    out_specs=pl.BlockSpec((D_FEAT, _BN), lambda i: (0, i)),
    out_shape=jax.ShapeDtypeStruct((D_FEAT, N_NODES), jnp.float32),
)


def kernel(x, synset_indices, synset_values):
    xt = x.T
    pad = EPT_PAD - EDGES_PER_TILE
    row = synset_indices[0].reshape(NW, EDGES_PER_TILE)
    col = synset_indices[1].reshape(NW, EDGES_PER_TILE)
    val = synset_values.reshape(NW, EDGES_PER_TILE)
    row = jnp.pad(row, ((0, 0), (0, pad))).reshape(NW, NCHUNK, CHUNK)
    col = jnp.pad(col, ((0, 0), (0, pad))).reshape(NW, NCHUNK, CHUNK)
    val = jnp.pad(val, ((0, 0), (0, pad))).reshape(NW, NCHUNK, CHUNK)
    part = _sc_spmm(xt, col, row, val)
    return _merge(part)
